# Initial kernel scaffold; baseline (speedup 1.0000x reference)
import functools

import jax
import jax.numpy as jnp
from jax import lax
from jax.experimental import pallas as pl
from jax.experimental.pallas import tpu as pltpu
from jax.experimental.pallas import tpu_sc as plsc

B, N, F = 16, 4096, 64
L = 16
C = N // L
NB = 256
PASSES = (0, 8, 16, 24)


def _keys_body(px_ref, py_ref, pz_ref, out_ref):
    x = px_ref[...]
    y = py_ref[...]
    z = pz_ref[...]
    m = jnp.sqrt(x * x + y * y + z * z)
    bits = lax.bitcast_convert_type(m, jnp.int32)
    out_ref[...] = bits ^ jnp.int32(-1)


def _compute_keys(px, py, pz):
    return pl.pallas_call(
        _keys_body,
        out_shape=jax.ShapeDtypeStruct((B, N), jnp.int32),
    )(px, py, pz)


def _sc_body(keys_hbm, px_hbm, py_hbm, pz_hbm, e_hbm,
             opx_hbm, opy_hbm, opz_hbm, oe_hbm,
             knat, ka, va, kb, vb, hist, totals, d0, d1, o0, o1):
    c = lax.axis_index("c")
    s = lax.axis_index("s")
    row = s
    iota = lax.iota(jnp.int32, L)
    ones = jnp.full((L,), 1, jnp.int32)
    zeros = jnp.full((L,), 0, jnp.int32)

    pltpu.sync_copy(keys_hbm.at[row], knat)

    @pl.when(c == 0)
    def _():
        pltpu.sync_copy(px_hbm.at[row], d0)
        pltpu.sync_copy(py_hbm.at[row], d1)

    @pl.when(c == 1)
    def _():
        pltpu.sync_copy(pz_hbm.at[row], d0)
        pltpu.sync_copy(e_hbm.at[row], d1)

    def vrow(t):
        return pl.ds(pl.multiple_of(t * L, L), L)

    def fill(t, _):
        k = knat[vrow(t)]
        e = t * L + iota
        phys = ((e & (C - 1)) << 4) + lax.shift_right_logical(e, 8)
        plsc.store_scatter(ka, [phys], k)
        plsc.store_scatter(va, [phys], e)
        return _

    lax.fori_loop(0, C, fill, 0, unroll=2)

    bufs = [(ka, va, kb, vb), (kb, vb, ka, va)]
    for p, sh in enumerate(PASSES):
        kin, vin, kout, vout = bufs[p % 2]

        def zero(t, _):
            hist[vrow(t)] = zeros
            return _

        lax.fori_loop(0, NB, zero, 0, unroll=4)

        def count(t, _, kin=kin, sh=sh):
            k = kin[vrow(t)]
            d = lax.shift_right_logical(k, sh) & (NB - 1)
            plsc.addupdate_scatter(hist, [(d << 4) + iota], ones)
            return _

        lax.fori_loop(0, C, count, 0, unroll=2)

        def bucket_totals(t, _):
            totals[t] = jnp.sum(hist[vrow(t)])
            return _

        lax.fori_loop(0, NB, bucket_totals, 0, unroll=2)

        def scan_totals(t, carry):
            tv = totals[vrow(t)]
            cs = plsc.cumsum(tv)
            totals[vrow(t)] = cs - tv + carry
            return carry + jnp.sum(tv)

        lax.fori_loop(0, L, scan_totals, jnp.int32(0))

        def bucket_offsets(d, _):
            rowv = hist[vrow(d)]
            cs = plsc.cumsum(rowv)
            hist[vrow(d)] = totals[d] + cs - rowv
            return _

        lax.fori_loop(0, NB, bucket_offsets, 0, unroll=2)

        def permute(t, _, kin=kin, vin=vin, kout=kout, vout=vout, sh=sh):
            k = kin[vrow(t)]
            v = vin[vrow(t)]
            d = lax.shift_right_logical(k, sh) & (NB - 1)
            a = (d << 4) + iota
            cur = plsc.load_gather(hist, [a])
            plsc.store_scatter(hist, [a], cur + ones)
            phys = ((cur & (C - 1)) << 4) + lax.shift_right_logical(cur, 8)
            plsc.store_scatter(kout, [phys], k)
            plsc.store_scatter(vout, [phys], v)
            return _

        lax.fori_loop(0, C, permute, 0, unroll=2)

    def out_gather(t, _):
        pp = t * L + iota
        phys = ((pp & (C - 1)) << 4) + lax.shift_right_logical(pp, 8)
        idxv = plsc.load_gather(va, [phys])
        o0[vrow(t)] = plsc.load_gather(d0, [idxv])
        o1[vrow(t)] = plsc.load_gather(d1, [idxv])
        return _

    lax.fori_loop(0, C, out_gather, 0, unroll=2)

    @pl.when(c == 0)
    def _():
        pltpu.sync_copy(o0, opx_hbm.at[row])
        pltpu.sync_copy(o1, opy_hbm.at[row])

    @pl.when(c == 1)
    def _():
        pltpu.sync_copy(o0, opz_hbm.at[row])
        pltpu.sync_copy(o1, oe_hbm.at[row])


def _sc_sort(keys, px, py, pz, e):
    fvec = jax.ShapeDtypeStruct((B, N), jnp.float32)
    fn = pl.kernel(
        _sc_body,
        mesh=plsc.VectorSubcoreMesh(core_axis_name="c", subcore_axis_name="s"),
        out_type=[fvec, fvec, fvec, fvec],
        scratch_types=[
            pltpu.VMEM((N,), jnp.int32),
            pltpu.VMEM((N,), jnp.int32),
            pltpu.VMEM((N,), jnp.int32),
            pltpu.VMEM((N,), jnp.int32),
            pltpu.VMEM((N,), jnp.int32),
            pltpu.VMEM((N,), jnp.int32),
            pltpu.VMEM((NB,), jnp.int32),
            pltpu.VMEM((N,), jnp.float32),
            pltpu.VMEM((N,), jnp.float32),
            pltpu.VMEM((N,), jnp.float32),
            pltpu.VMEM((N,), jnp.float32),
        ],
    )
    return fn(keys, px, py, pz, e)


def kernel(dau_px, dau_py, dau_pz, dau_e, jet_features):
    keys = _compute_keys(dau_px, dau_py, dau_pz)
    opx, opy, opz, oe = _sc_sort(keys, dau_px, dau_py, dau_pz, dau_e)
    return opx, opy, opz, oe, jet_features


# speedup vs baseline: 1.3680x; 1.3680x over previous
import functools

import jax
import jax.numpy as jnp
from jax import lax
from jax.experimental import pallas as pl
from jax.experimental.pallas import tpu as pltpu
from jax.experimental.pallas import tpu_sc as plsc

B, N, F = 16, 4096, 64
L = 16
C = N // L
NB = 256
PASSES = (0, 8, 16, 24)


def _keys_body(px_ref, py_ref, pz_ref, out_ref):
    x = px_ref[...]
    y = py_ref[...]
    z = pz_ref[...]
    m = jnp.sqrt(x * x + y * y + z * z)
    bits = lax.bitcast_convert_type(m, jnp.int32)
    out_ref[...] = bits ^ jnp.int32(-1)


def _compute_keys(px, py, pz):
    return pl.pallas_call(
        _keys_body,
        out_shape=jax.ShapeDtypeStruct((B, N), jnp.int32),
    )(px, py, pz)


def _sc_body(keys_hbm, data_hbm, out_hbm,
             knat, ka, va, kb, vb, hist, d0, d1, o0, o1):
    c = lax.axis_index("c")
    s = lax.axis_index("s")
    row = s
    a0 = 2 * c
    a1 = 2 * c + 1
    iota = lax.iota(jnp.int32, L)
    ones = jnp.full((L,), 1, jnp.int32)
    zeros = jnp.full((L,), 0, jnp.int32)

    pltpu.sync_copy(keys_hbm.at[row], knat)
    pltpu.sync_copy(data_hbm.at[a0, row], d0)
    pltpu.sync_copy(data_hbm.at[a1, row], d1)

    def vrow(t):
        return pl.ds(pl.multiple_of(t * L, L), L)

    def fill(t, _):
        k = knat[vrow(t)]
        e = t * L + iota
        phys = ((e & (C - 1)) << 4) + lax.shift_right_logical(e, 8)
        plsc.store_scatter(ka, [phys], k)
        plsc.store_scatter(va, [phys], e)
        return _

    lax.fori_loop(0, C, fill, 0, unroll=2)

    bufs = [(ka, va, kb, vb), (kb, vb, ka, va)]
    for p, sh in enumerate(PASSES):
        kin, vin, kout, vout = bufs[p % 2]

        def zero(t, _):
            hist[vrow(t)] = zeros
            return _

        lax.fori_loop(0, NB, zero, 0, unroll=4)

        def count(t, _, kin=kin, sh=sh):
            k = kin[vrow(t)]
            d = lax.shift_right_logical(k, sh) & (NB - 1)
            plsc.addupdate_scatter(hist, [(d << 4) + iota], ones)
            return _

        lax.fori_loop(0, C, count, 0, unroll=2)

        def bucket_offsets(d, carry):
            rowv = hist[vrow(d)]
            cs = plsc.cumsum(rowv)
            hist[vrow(d)] = carry + cs - rowv
            return carry + jnp.sum(rowv)

        lax.fori_loop(0, NB, bucket_offsets, jnp.int32(0), unroll=2)

        def permute(t, _, kin=kin, vin=vin, kout=kout, vout=vout, sh=sh):
            k = kin[vrow(t)]
            v = vin[vrow(t)]
            d = lax.shift_right_logical(k, sh) & (NB - 1)
            a = (d << 4) + iota
            cur = plsc.load_gather(hist, [a])
            plsc.store_scatter(hist, [a], cur + ones)
            phys = ((cur & (C - 1)) << 4) + lax.shift_right_logical(cur, 8)
            plsc.store_scatter(kout, [phys], k)
            plsc.store_scatter(vout, [phys], v)
            return _

        lax.fori_loop(0, C, permute, 0, unroll=2)

    def out_gather(t, _):
        pp = t * L + iota
        phys = ((pp & (C - 1)) << 4) + lax.shift_right_logical(pp, 8)
        idxv = plsc.load_gather(va, [phys])
        o0[vrow(t)] = plsc.load_gather(d0, [idxv])
        o1[vrow(t)] = plsc.load_gather(d1, [idxv])
        return _

    lax.fori_loop(0, C, out_gather, 0, unroll=2)

    pltpu.sync_copy(o0, out_hbm.at[a0, row])
    pltpu.sync_copy(o1, out_hbm.at[a1, row])


def _sc_sort(keys, data):
    fn = pl.kernel(
        _sc_body,
        mesh=plsc.VectorSubcoreMesh(core_axis_name="c", subcore_axis_name="s"),
        out_type=[jax.ShapeDtypeStruct((4, B, N), jnp.float32)],
        compiler_params=pltpu.CompilerParams(needs_layout_passes=False),
        scratch_types=[
            pltpu.VMEM((N,), jnp.int32),
            pltpu.VMEM((N,), jnp.int32),
            pltpu.VMEM((N,), jnp.int32),
            pltpu.VMEM((N,), jnp.int32),
            pltpu.VMEM((N,), jnp.int32),
            pltpu.VMEM((N,), jnp.int32),
            pltpu.VMEM((N,), jnp.float32),
            pltpu.VMEM((N,), jnp.float32),
            pltpu.VMEM((N,), jnp.float32),
            pltpu.VMEM((N,), jnp.float32),
        ],
    )
    return fn(keys, data)


def kernel(dau_px, dau_py, dau_pz, dau_e, jet_features):
    keys = _compute_keys(dau_px, dau_py, dau_pz)
    data = jnp.stack([dau_px, dau_py, dau_pz, dau_e])
    (out,) = _sc_sort(keys, data)
    return out[0], out[1], out[2], out[3], jet_features
